# baseline (speedup 1.0000x reference)
import functools
import math

import numpy as np
import jax
import jax.numpy as jnp
from jax.experimental import pallas as pl
from jax.experimental.pallas import tpu as pltpu

_NEG = -3.4e38


def _cnt_transposed(L_Q: int, L_K: int, sample_k: int) -> jax.Array:
    idx = jax.random.randint(jax.random.key(42), (L_Q, sample_k), 0, L_K)
    cnt = jnp.zeros((L_K, L_Q), jnp.int8)
    ql = jnp.broadcast_to(jnp.arange(L_Q)[:, None], (L_Q, sample_k))
    return cnt.at[idx, ql].add(1)


def _make_body(L: int, D: int, U: int, KB: int, scale: float):
    def body(cnt_ref, q_ref, k_ref, v_ref, ctx_ref, attn_ref, oh_ref):
        q = q_ref[0]

        mx = jnp.full((1, L), _NEG, jnp.float32)
        sm = jnp.zeros((1, L), jnp.float32)
        for kb in range(L // KB):
            k_blk = k_ref[0, kb * KB:(kb + 1) * KB, :]
            st = jax.lax.dot_general(
                k_blk, q, (((1,), (1,)), ((), ())),
                preferred_element_type=jnp.float32)
            cf = cnt_ref[kb * KB:(kb + 1) * KB, :].astype(jnp.float32)
            masked = jnp.where(cf > 0, st, _NEG)
            mx = jnp.maximum(mx, jnp.max(masked, axis=0, keepdims=True))
            sm = sm + jnp.sum(st * cf, axis=0, keepdims=True)
        M = mx - sm * (1.0 / L)

        lane = jax.lax.broadcasted_iota(jnp.int32, (1, L), 1)
        for u in range(U):
            m0 = jnp.max(M, axis=1, keepdims=True)
            i0 = jnp.min(jnp.where(M == m0, lane, L), axis=1,
                         keepdims=True)
            sel = lane == i0
            oh_ref[u:u + 1, :] = sel.astype(jnp.float32)
            M = jnp.where(sel, _NEG, M)

        qr = jax.lax.dot_general(
            oh_ref[...], q, (((1,), (0,)), ((), ())),
            preferred_element_type=jnp.float32)
        qk = jax.lax.dot_general(
            qr, k_ref[0], (((1,), (1,)), ((), ())),
            preferred_element_type=jnp.float32)
        rowi = jax.lax.broadcasted_iota(jnp.int32, (U, L), 0)
        coli = jax.lax.broadcasted_iota(jnp.int32, (U, L), 1)
        s = jnp.where(coli <= rowi, qk * scale, -1000000000.0)
        smax = jnp.max(s, axis=1, keepdims=True)
        e = jnp.exp(s - smax)
        attn = e / jnp.sum(e, axis=1, keepdims=True)
        attn_ref[0] = attn
        ctx_ref[0] = jax.lax.dot_general(
            attn, v_ref[0], (((1,), (0,)), ((), ())),
            preferred_element_type=jnp.float32)

    return body


def kernel(queries, keys, values):
    B, L, H, D = queries.shape
    BH = B * H
    U = 5 * int(math.ceil(math.log(float(L))))
    scale = 1.0 / math.sqrt(D)
    KB = 512

    q = queries.reshape(BH, L, D)
    k = keys.reshape(BH, L, D)
    v = values.reshape(BH, L, D)
    cnt_t = _cnt_transposed(L, L, U)

    ctx, attn = pl.pallas_call(
        _make_body(L, D, U, KB, scale),
        grid=(BH,),
        in_specs=[
            pl.BlockSpec((L, L), lambda i: (0, 0)),
            pl.BlockSpec((1, L, D), lambda i: (i, 0, 0)),
            pl.BlockSpec((1, L, D), lambda i: (i, 0, 0)),
            pl.BlockSpec((1, L, D), lambda i: (i, 0, 0)),
        ],
        out_specs=[
            pl.BlockSpec((1, U, D), lambda i: (i, 0, 0)),
            pl.BlockSpec((1, U, L), lambda i: (i, 0, 0)),
        ],
        out_shape=[
            jax.ShapeDtypeStruct((BH, U, D), jnp.float32),
            jax.ShapeDtypeStruct((BH, U, L), jnp.float32),
        ],
        scratch_shapes=[pltpu.VMEM((U, L), jnp.float32)],
    )(cnt_t, q, k, v)

    return ctx.reshape(B, H, U, D), attn.reshape(B, H, U, L)


# speedup vs baseline: 2.1794x; 2.1794x over previous
import functools
import math

import numpy as np
import jax
import jax.numpy as jnp
from jax.experimental import pallas as pl
from jax.experimental.pallas import tpu as pltpu

_NEG = -3.4e38


@functools.lru_cache(maxsize=None)
def _sample_constants(L_Q: int, L_K: int, sample_k: int):
    with jax.ensure_compile_time_eval():
        idx = np.asarray(
            jax.random.randint(jax.random.key(42), (L_Q, sample_k), 0, L_K)
        )
    cnt = np.zeros((L_K, L_Q), np.float32)
    np.add.at(cnt, (idx, np.arange(L_Q)[:, None]), 1.0)
    logmask = np.where(cnt > 0, 0.0, _NEG).astype(np.float32)
    return logmask, cnt


def _make_body(L: int, D: int, U: int, KB: int, scale: float):
    LB = L // 8

    def body(lm_ref, cf_ref, q_ref, k_ref, v_ref, ctx_ref, attn_ref, oh_ref):
        q = q_ref[0]

        mx = jnp.full((1, L), _NEG, jnp.float32)
        sm = jnp.zeros((1, L), jnp.float32)
        for kb in range(L // KB):
            k_blk = k_ref[0, kb * KB:(kb + 1) * KB, :]
            st = jax.lax.dot_general(
                k_blk, q, (((1,), (1,)), ((), ())),
                preferred_element_type=jnp.float32)
            lm = lm_ref[kb * KB:(kb + 1) * KB, :]
            cf = cf_ref[kb * KB:(kb + 1) * KB, :]
            mx = jnp.maximum(mx, jnp.max(st + lm, axis=0, keepdims=True))
            sm = sm + jnp.sum(st * cf, axis=0, keepdims=True)
        M = mx - sm * (1.0 / L)

        M8 = M.reshape(8, LB)
        fi = (jax.lax.broadcasted_iota(jnp.int32, (8, LB), 0) * LB
              + jax.lax.broadcasted_iota(jnp.int32, (8, LB), 1))
        lane = jax.lax.broadcasted_iota(jnp.int32, (1, L), 1)
        for u in range(U):
            m0 = jnp.max(M8, axis=(0, 1), keepdims=True)
            i0 = jnp.min(jnp.where(M8 == m0, fi, L), axis=(0, 1),
                         keepdims=True)
            oh_ref[u:u + 1, :] = (lane == i0).astype(jnp.float32)
            M8 = jnp.where(fi == i0, _NEG, M8)

        qr = jax.lax.dot_general(
            oh_ref[...], q, (((1,), (0,)), ((), ())),
            preferred_element_type=jnp.float32)
        qk = jax.lax.dot_general(
            qr, k_ref[0], (((1,), (1,)), ((), ())),
            preferred_element_type=jnp.float32)
        rowi = jax.lax.broadcasted_iota(jnp.int32, (U, L), 0)
        coli = jax.lax.broadcasted_iota(jnp.int32, (U, L), 1)
        s = jnp.where(coli <= rowi, qk * scale, -1000000000.0)
        smax = jnp.max(s, axis=1, keepdims=True)
        e = jnp.exp(s - smax)
        attn = e / jnp.sum(e, axis=1, keepdims=True)
        attn_ref[0] = attn
        ctx_ref[0] = jax.lax.dot_general(
            attn, v_ref[0], (((1,), (0,)), ((), ())),
            preferred_element_type=jnp.float32)

    return body


def kernel(queries, keys, values):
    B, L, H, D = queries.shape
    BH = B * H
    U = 5 * int(math.ceil(math.log(float(L))))
    scale = 1.0 / math.sqrt(D)
    KB = 512

    q = queries.reshape(BH, L, D)
    k = keys.reshape(BH, L, D)
    v = values.reshape(BH, L, D)
    lm_np, cf_np = _sample_constants(L, L, U)
    lm_c = jnp.asarray(lm_np)
    cf_c = jnp.asarray(cf_np)

    ctx, attn = pl.pallas_call(
        _make_body(L, D, U, KB, scale),
        grid=(BH,),
        in_specs=[
            pl.BlockSpec((L, L), lambda i: (0, 0)),
            pl.BlockSpec((L, L), lambda i: (0, 0)),
            pl.BlockSpec((1, L, D), lambda i: (i, 0, 0)),
            pl.BlockSpec((1, L, D), lambda i: (i, 0, 0)),
            pl.BlockSpec((1, L, D), lambda i: (i, 0, 0)),
        ],
        out_specs=[
            pl.BlockSpec((1, U, D), lambda i: (i, 0, 0)),
            pl.BlockSpec((1, U, L), lambda i: (i, 0, 0)),
        ],
        out_shape=[
            jax.ShapeDtypeStruct((BH, U, D), jnp.float32),
            jax.ShapeDtypeStruct((BH, U, L), jnp.float32),
        ],
        scratch_shapes=[pltpu.VMEM((U, L), jnp.float32)],
    )(lm_c, cf_c, q, k, v)

    return ctx.reshape(B, H, U, D), attn.reshape(B, H, U, L)


# speedup vs baseline: 2.2999x; 1.0553x over previous
import functools
import math

import numpy as np
import jax
import jax.numpy as jnp
from jax.experimental import pallas as pl
from jax.experimental.pallas import tpu as pltpu

_NEG = -3.4e38


@functools.lru_cache(maxsize=None)
def _cnt_transposed(L_Q: int, L_K: int, sample_k: int) -> np.ndarray:
    with jax.ensure_compile_time_eval():
        idx = np.asarray(
            jax.random.randint(jax.random.key(42), (L_Q, sample_k), 0, L_K)
        )
    cnt = np.zeros((L_K, L_Q), np.int8)
    np.add.at(cnt, (idx, np.arange(L_Q)[:, None]), 1)
    return cnt


def _make_body(L: int, D: int, U: int, KB: int, scale: float, HPB: int):
    def body(cnt_ref, q_ref, k_ref, v_ref, ctx_ref, attn_ref, oh_ref):
        lane = jax.lax.broadcasted_iota(jnp.int32, (1, L), 1)
        rowi = jax.lax.broadcasted_iota(jnp.int32, (U, L), 0)
        coli = jax.lax.broadcasted_iota(jnp.int32, (U, L), 1)

        for hh in range(HPB):
            q = q_ref[hh]

            mx = jnp.full((1, L), _NEG, jnp.float32)
            sm = jnp.zeros((1, L), jnp.float32)
            for kb in range(L // KB):
                k_blk = k_ref[hh, kb * KB:(kb + 1) * KB, :]
                st = jax.lax.dot_general(
                    k_blk, q, (((1,), (1,)), ((), ())),
                    preferred_element_type=jnp.float32)
                cf = cnt_ref[kb * KB:(kb + 1) * KB, :].astype(jnp.float32)
                masked = jnp.where(cf > 0, st, _NEG)
                mx = jnp.maximum(mx, jnp.max(masked, axis=0, keepdims=True))
                sm = sm + jnp.sum(st * cf, axis=0, keepdims=True)
            M = mx - sm * (1.0 / L)

            for u in range(U):
                m0 = jnp.max(M, axis=1, keepdims=True)
                i0 = jnp.min(jnp.where(M == m0, lane, L), axis=1,
                             keepdims=True)
                sel = lane == i0
                oh_ref[hh, u:u + 1, :] = sel.astype(jnp.float32)
                M = jnp.where(sel, _NEG, M)

            qr = jax.lax.dot_general(
                oh_ref[hh], q, (((1,), (0,)), ((), ())),
                preferred_element_type=jnp.float32)
            qk = jax.lax.dot_general(
                qr, k_ref[hh], (((1,), (1,)), ((), ())),
                preferred_element_type=jnp.float32)
            s = jnp.where(coli <= rowi, qk * scale, -1000000000.0)
            smax = jnp.max(s, axis=1, keepdims=True)
            e = jnp.exp(s - smax)
            attn = e / jnp.sum(e, axis=1, keepdims=True)
            attn_ref[hh] = attn
            ctx_ref[hh] = jax.lax.dot_general(
                attn, v_ref[hh], (((1,), (0,)), ((), ())),
                preferred_element_type=jnp.float32)

    return body


def kernel(queries, keys, values):
    B, L, H, D = queries.shape
    BH = B * H
    U = 5 * int(math.ceil(math.log(float(L))))
    scale = 1.0 / math.sqrt(D)
    KB = 512
    HPB = 2

    q = queries.reshape(BH, L, D)
    k = keys.reshape(BH, L, D)
    v = values.reshape(BH, L, D)
    cnt_t = jnp.asarray(_cnt_transposed(L, L, U))

    ctx, attn = pl.pallas_call(
        _make_body(L, D, U, KB, scale, HPB),
        grid=(BH // HPB,),
        in_specs=[
            pl.BlockSpec((L, L), lambda i: (0, 0)),
            pl.BlockSpec((HPB, L, D), lambda i: (i, 0, 0)),
            pl.BlockSpec((HPB, L, D), lambda i: (i, 0, 0)),
            pl.BlockSpec((HPB, L, D), lambda i: (i, 0, 0)),
        ],
        out_specs=[
            pl.BlockSpec((HPB, U, D), lambda i: (i, 0, 0)),
            pl.BlockSpec((HPB, U, L), lambda i: (i, 0, 0)),
        ],
        out_shape=[
            jax.ShapeDtypeStruct((BH, U, D), jnp.float32),
            jax.ShapeDtypeStruct((BH, U, L), jnp.float32),
        ],
        scratch_shapes=[pltpu.VMEM((HPB, U, L), jnp.float32)],
    )(cnt_t, q, k, v)

    return ctx.reshape(B, H, U, D), attn.reshape(B, H, U, L)


# speedup vs baseline: 2.7601x; 1.2001x over previous
import functools
import math

import numpy as np
import jax
import jax.numpy as jnp
from jax.experimental import pallas as pl
from jax.experimental.pallas import tpu as pltpu

_NEG = -3.4e38


@functools.lru_cache(maxsize=None)
def _cnt_transposed(L_Q: int, L_K: int, sample_k: int) -> np.ndarray:
    with jax.ensure_compile_time_eval():
        idx = np.asarray(
            jax.random.randint(jax.random.key(42), (L_Q, sample_k), 0, L_K)
        )
    cnt = np.zeros((L_K, L_Q), np.int8)
    np.add.at(cnt, (idx, np.arange(L_Q)[:, None]), 1)
    return cnt


def _make_body(L: int, D: int, U: int, KB: int, scale: float, HPB: int):
    def body(cnt_ref, q_ref, k_ref, v_ref, ctx_ref, attn_ref, oh_ref):
        lane = jax.lax.broadcasted_iota(jnp.int32, (1, L), 1)
        rowi = jax.lax.broadcasted_iota(jnp.int32, (U, L), 0)
        coli = jax.lax.broadcasted_iota(jnp.int32, (U, L), 1)

        for hh in range(HPB):
            q = q_ref[hh]

            mx = jnp.full((1, L), _NEG, jnp.float32)
            sm = jnp.zeros((1, L), jnp.float32)
            for kb in range(L // KB):
                k_blk = k_ref[hh, kb * KB:(kb + 1) * KB, :]
                st = jax.lax.dot_general(
                    k_blk, q, (((1,), (1,)), ((), ())),
                    preferred_element_type=jnp.float32)
                cf = cnt_ref[kb * KB:(kb + 1) * KB, :].astype(jnp.float32)
                masked = jnp.where(cf > 0, st, _NEG)
                mx = jnp.maximum(mx, jnp.max(masked, axis=0, keepdims=True))
                sm = sm + jnp.sum(st * cf, axis=0, keepdims=True)
            M = mx - sm * (1.0 / L)

            for u in range(U):
                i0 = jnp.argmax(M, axis=1).reshape(1, 1)
                sel = lane == i0
                oh_ref[hh, u:u + 1, :] = sel.astype(jnp.float32)
                M = jnp.where(sel, _NEG, M)

            qr = jax.lax.dot_general(
                oh_ref[hh], q, (((1,), (0,)), ((), ())),
                preferred_element_type=jnp.float32)
            qk = jax.lax.dot_general(
                qr, k_ref[hh], (((1,), (1,)), ((), ())),
                preferred_element_type=jnp.float32)
            s = jnp.where(coli <= rowi, qk * scale, -1000000000.0)
            smax = jnp.max(s, axis=1, keepdims=True)
            e = jnp.exp(s - smax)
            attn = e / jnp.sum(e, axis=1, keepdims=True)
            attn_ref[hh] = attn
            ctx_ref[hh] = jax.lax.dot_general(
                attn, v_ref[hh], (((1,), (0,)), ((), ())),
                preferred_element_type=jnp.float32)

    return body


def kernel(queries, keys, values):
    B, L, H, D = queries.shape
    BH = B * H
    U = 5 * int(math.ceil(math.log(float(L))))
    scale = 1.0 / math.sqrt(D)
    KB = 512
    HPB = 2

    q = queries.reshape(BH, L, D)
    k = keys.reshape(BH, L, D)
    v = values.reshape(BH, L, D)
    cnt_t = jnp.asarray(_cnt_transposed(L, L, U))

    ctx, attn = pl.pallas_call(
        _make_body(L, D, U, KB, scale, HPB),
        grid=(BH // HPB,),
        in_specs=[
            pl.BlockSpec((L, L), lambda i: (0, 0)),
            pl.BlockSpec((HPB, L, D), lambda i: (i, 0, 0)),
            pl.BlockSpec((HPB, L, D), lambda i: (i, 0, 0)),
            pl.BlockSpec((HPB, L, D), lambda i: (i, 0, 0)),
        ],
        out_specs=[
            pl.BlockSpec((HPB, U, D), lambda i: (i, 0, 0)),
            pl.BlockSpec((HPB, U, L), lambda i: (i, 0, 0)),
        ],
        out_shape=[
            jax.ShapeDtypeStruct((BH, U, D), jnp.float32),
            jax.ShapeDtypeStruct((BH, U, L), jnp.float32),
        ],
        scratch_shapes=[pltpu.VMEM((HPB, U, L), jnp.float32)],
    )(cnt_t, q, k, v)

    return ctx.reshape(B, H, U, D), attn.reshape(B, H, U, L)


# speedup vs baseline: 3.6191x; 1.3112x over previous
import functools
import math

import numpy as np
import jax
import jax.numpy as jnp
from jax.experimental import pallas as pl
from jax.experimental.pallas import tpu as pltpu

_NEG = -3.4e38


@functools.lru_cache(maxsize=None)
def _cnt_transposed(L_Q: int, L_K: int, sample_k: int) -> np.ndarray:
    with jax.ensure_compile_time_eval():
        idx = np.asarray(
            jax.random.randint(jax.random.key(42), (L_Q, sample_k), 0, L_K)
        )
    cnt = np.zeros((L_K, L_Q), np.int8)
    np.add.at(cnt, (idx, np.arange(L_Q)[:, None]), 1)
    return cnt


def _make_body(L: int, D: int, U: int, KB: int, scale: float, HPB: int):
    def body(cnt_ref, q_ref, k_ref, v_ref, ctx_ref, attn_ref, oh_ref):
        lane = jax.lax.broadcasted_iota(jnp.int32, (1, L), 1)
        rowi = jax.lax.broadcasted_iota(jnp.int32, (U, L), 0)
        coli = jax.lax.broadcasted_iota(jnp.int32, (U, L), 1)

        Ms = []
        for hh in range(HPB):
            q = q_ref[hh]
            mx = jnp.full((1, L), _NEG, jnp.float32)
            sm = jnp.zeros((1, L), jnp.float32)
            for kb in range(L // KB):
                k_blk = k_ref[hh, kb * KB:(kb + 1) * KB, :]
                st = jax.lax.dot_general(
                    k_blk, q, (((1,), (1,)), ((), ())),
                    preferred_element_type=jnp.float32)
                cf = cnt_ref[kb * KB:(kb + 1) * KB, :].astype(jnp.float32)
                masked = jnp.where(cf > 0, st, _NEG)
                mx = jnp.maximum(mx, jnp.max(masked, axis=0, keepdims=True))
                sm = sm + jnp.sum(st * cf, axis=0, keepdims=True)
            Ms.append(mx - sm * (1.0 / L))

        for u in range(U):
            for hh in range(HPB):
                i0 = jnp.argmax(Ms[hh], axis=1).reshape(1, 1)
                sel = lane == i0
                oh_ref[hh, u:u + 1, :] = sel.astype(jnp.float32)
                Ms[hh] = jnp.where(sel, _NEG, Ms[hh])

        for hh in range(HPB):
            qr = jax.lax.dot_general(
                oh_ref[hh], q_ref[hh], (((1,), (0,)), ((), ())),
                preferred_element_type=jnp.float32)
            qk = jax.lax.dot_general(
                qr, k_ref[hh], (((1,), (1,)), ((), ())),
                preferred_element_type=jnp.float32)
            s = jnp.where(coli <= rowi, qk * scale, -1000000000.0)
            smax = jnp.max(s, axis=1, keepdims=True)
            e = jnp.exp(s - smax)
            attn = e / jnp.sum(e, axis=1, keepdims=True)
            attn_ref[hh] = attn
            ctx_ref[hh] = jax.lax.dot_general(
                attn, v_ref[hh], (((1,), (0,)), ((), ())),
                preferred_element_type=jnp.float32)

    return body


def kernel(queries, keys, values):
    B, L, H, D = queries.shape
    BH = B * H
    U = 5 * int(math.ceil(math.log(float(L))))
    scale = 1.0 / math.sqrt(D)
    KB = 512
    HPB = 2

    q = queries.reshape(BH, L, D)
    k = keys.reshape(BH, L, D)
    v = values.reshape(BH, L, D)
    cnt_t = jnp.asarray(_cnt_transposed(L, L, U))

    ctx, attn = pl.pallas_call(
        _make_body(L, D, U, KB, scale, HPB),
        grid=(BH // HPB,),
        in_specs=[
            pl.BlockSpec((L, L), lambda i: (0, 0)),
            pl.BlockSpec((HPB, L, D), lambda i: (i, 0, 0)),
            pl.BlockSpec((HPB, L, D), lambda i: (i, 0, 0)),
            pl.BlockSpec((HPB, L, D), lambda i: (i, 0, 0)),
        ],
        out_specs=[
            pl.BlockSpec((HPB, U, D), lambda i: (i, 0, 0)),
            pl.BlockSpec((HPB, U, L), lambda i: (i, 0, 0)),
        ],
        out_shape=[
            jax.ShapeDtypeStruct((BH, U, D), jnp.float32),
            jax.ShapeDtypeStruct((BH, U, L), jnp.float32),
        ],
        scratch_shapes=[pltpu.VMEM((HPB, U, L), jnp.float32)],
    )(cnt_t, q, k, v)

    return ctx.reshape(B, H, U, D), attn.reshape(B, H, U, L)


# speedup vs baseline: 4.0741x; 1.1257x over previous
import functools
import math

import numpy as np
import jax
import jax.numpy as jnp
from jax.experimental import pallas as pl
from jax.experimental.pallas import tpu as pltpu

_NEG = -3.4e38


@functools.lru_cache(maxsize=None)
def _cnt_transposed(L_Q: int, L_K: int, sample_k: int) -> np.ndarray:
    with jax.ensure_compile_time_eval():
        idx = np.asarray(
            jax.random.randint(jax.random.key(42), (L_Q, sample_k), 0, L_K)
        )
    cnt = np.zeros((L_K, L_Q), np.int8)
    np.add.at(cnt, (idx, np.arange(L_Q)[:, None]), 1)
    return cnt


def _make_body(L: int, D: int, U: int, KB: int, scale: float, HPB: int):
    def body(cnt_ref, q_ref, k_ref, v_ref, ctx_ref, attn_ref, oh_ref):
        lane = jax.lax.broadcasted_iota(jnp.int32, (1, L), 1)
        rowi = jax.lax.broadcasted_iota(jnp.int32, (U, L), 0)
        coli = jax.lax.broadcasted_iota(jnp.int32, (U, L), 1)

        Ms = []
        for hh in range(HPB):
            q = q_ref[hh]
            mx = jnp.full((1, L), _NEG, jnp.float32)
            sm = jnp.zeros((1, L), jnp.float32)
            for kb in range(L // KB):
                k_blk = k_ref[hh, kb * KB:(kb + 1) * KB, :]
                st = jax.lax.dot_general(
                    k_blk, q, (((1,), (1,)), ((), ())),
                    preferred_element_type=jnp.float32)
                cf = cnt_ref[kb * KB:(kb + 1) * KB, :].astype(jnp.float32)
                masked = jnp.where(cf > 0, st, _NEG)
                mx = jnp.maximum(mx, jnp.max(masked, axis=0, keepdims=True))
                sm = sm + jnp.sum(st * cf, axis=0, keepdims=True)
            Ms.append(mx - sm * (1.0 / L))

        for u in range(U):
            for hh in range(HPB):
                i0 = jnp.argmax(Ms[hh], axis=1).reshape(1, 1)
                sel = lane == i0
                oh_ref[hh, u:u + 1, :] = sel.astype(jnp.float32)
                Ms[hh] = jnp.where(sel, _NEG, Ms[hh])

        for hh in range(HPB):
            qr = jax.lax.dot_general(
                oh_ref[hh], q_ref[hh], (((1,), (0,)), ((), ())),
                preferred_element_type=jnp.float32)
            qk = jax.lax.dot_general(
                qr, k_ref[hh], (((1,), (1,)), ((), ())),
                preferred_element_type=jnp.float32)
            s = jnp.where(coli <= rowi, qk * scale, -1000000000.0)
            smax = jnp.max(s, axis=1, keepdims=True)
            e = jnp.exp(s - smax)
            attn = e / jnp.sum(e, axis=1, keepdims=True)
            attn_ref[hh] = attn
            ctx_ref[hh] = jax.lax.dot_general(
                attn, v_ref[hh], (((1,), (0,)), ((), ())),
                preferred_element_type=jnp.float32)

    return body


def kernel(queries, keys, values):
    B, L, H, D = queries.shape
    BH = B * H
    U = 5 * int(math.ceil(math.log(float(L))))
    scale = 1.0 / math.sqrt(D)
    KB = 512
    HPB = 4

    q = queries.reshape(BH, L, D)
    k = keys.reshape(BH, L, D)
    v = values.reshape(BH, L, D)
    cnt_t = jnp.asarray(_cnt_transposed(L, L, U))

    ctx, attn = pl.pallas_call(
        _make_body(L, D, U, KB, scale, HPB),
        grid=(BH // HPB,),
        in_specs=[
            pl.BlockSpec((L, L), lambda i: (0, 0)),
            pl.BlockSpec((HPB, L, D), lambda i: (i, 0, 0)),
            pl.BlockSpec((HPB, L, D), lambda i: (i, 0, 0)),
            pl.BlockSpec((HPB, L, D), lambda i: (i, 0, 0)),
        ],
        out_specs=[
            pl.BlockSpec((HPB, U, D), lambda i: (i, 0, 0)),
            pl.BlockSpec((HPB, U, L), lambda i: (i, 0, 0)),
        ],
        out_shape=[
            jax.ShapeDtypeStruct((BH, U, D), jnp.float32),
            jax.ShapeDtypeStruct((BH, U, L), jnp.float32),
        ],
        scratch_shapes=[pltpu.VMEM((HPB, U, L), jnp.float32)],
    )(cnt_t, q, k, v)

    return ctx.reshape(B, H, U, D), attn.reshape(B, H, U, L)
